# baseline (speedup 1.0000x reference)
import functools

import jax
import jax.numpy as jnp
from jax import lax
from jax.experimental import pallas as pl
from jax.experimental.pallas import tpu as pltpu
from jax.experimental.pallas import tpu_sc as plsc

N_NODES_K = 10000
N_EDGES_K = 160000
D_FEAT_K = 256
HALF_D = 128

CHUNK = 128
ROWS2D = 1280
PAD_E = ROWS2D * CHUNK - N_EDGES_K
NKS = ROWS2D // 16
NKD = ROWS2D // 32
NB = 2
ZROWS = 10112
ZTRASH = N_NODES_K
BLK = 200

_MESH = plsc.VectorSubcoreMesh(core_axis_name="c", subcore_axis_name="s")


@functools.partial(
    pl.kernel,
    out_type=jax.ShapeDtypeStruct((2, ZROWS, HALF_D), jnp.float32),
    mesh=_MESH,
    scratch_types=[
        pltpu.VMEM_SHARED((ZROWS, HALF_D), jnp.float32),
        pltpu.VMEM((NKD, CHUNK), jnp.int32),
        pltpu.VMEM((NKD, CHUNK), jnp.int32),
        pltpu.VMEM((CHUNK,), jnp.int32),
        pltpu.VMEM((CHUNK,), jnp.int32),
        pltpu.VMEM((CHUNK, HALF_D), jnp.float32),
        pltpu.VMEM((CHUNK, HALF_D), jnp.float32),
        pltpu.SemaphoreType.DMA,
        pltpu.SemaphoreType.DMA,
    ],
)
def _sc_degree(row_hbm, col_hbm, out_hbm, deg_sh, rbuf, cbuf, cs0, cs1, ones,
               zbuf, sem0, sem1):
    cid = lax.axis_index("c")
    sid = lax.axis_index("s")

    @pl.loop(0, CHUNK)
    def _(i):
        @pl.loop(0, HALF_D, step=16)
        def _(j):
            ones[i, pl.ds(j, 16)] = jnp.full((16,), 1.0, jnp.float32)
            zbuf[i, pl.ds(j, 16)] = jnp.zeros((16,), jnp.float32)

    base = sid * 632
    for part in range(4):
        pltpu.sync_copy(
            zbuf, deg_sh.at[pl.ds(base + part * CHUNK, CHUNK)])
    pltpu.sync_copy(
        zbuf.at[pl.ds(0, 120)], deg_sh.at[pl.ds(base + 512, 120)])
    plsc.subcore_barrier()

    base_r = cid * (16 * NKD) + sid * NKD
    pltpu.sync_copy(row_hbm.at[pl.ds(base_r, NKD)], rbuf)
    pltpu.sync_copy(col_hbm.at[pl.ds(base_r, NKD)], cbuf)

    trash16 = ZTRASH + lax.iota(jnp.int32, 16)
    slots = [(cs0, sem0), (cs1, sem1)]

    def _scat(b):
        cs, sem = slots[b]
        return pltpu.make_async_copy(ones, deg_sh.at[cs], sem)

    @pl.loop(0, NKD, step=NB)
    def _(k2):
        for b in range(NB):
            k = k2 + b
            cs, _sem = slots[b]

            @pl.when(k2 > 0)
            def _():
                _scat(b).wait()

            @pl.loop(0, CHUNK, step=16)
            def _(j):
                c16 = cbuf[k, pl.ds(j, 16)]
                r16 = rbuf[k, pl.ds(j, 16)]
                cs[pl.ds(j, 16)] = jnp.where(c16 == r16, trash16, c16)

            _scat(b).start(add=True)

    for b in range(NB):
        _scat(b).wait()

    plsc.subcore_barrier()
    pltpu.sync_copy(
        deg_sh.at[pl.ds(base, 632)],
        out_hbm.at[cid, pl.ds(base, 632)],
    )


@functools.partial(
    pl.kernel,
    out_type=jax.ShapeDtypeStruct((2, ZROWS, HALF_D), jnp.float32),
    mesh=_MESH,
    scratch_types=[
        pltpu.VMEM_SHARED((ZROWS, HALF_D), jnp.float32),
        pltpu.VMEM((2, CHUNK), jnp.int32),
        pltpu.VMEM((2, CHUNK), jnp.int32),
        pltpu.VMEM((CHUNK,), jnp.int32),
        pltpu.VMEM((CHUNK,), jnp.int32),
        pltpu.VMEM((CHUNK,), jnp.int32),
        pltpu.VMEM((CHUNK,), jnp.int32),
        pltpu.VMEM((CHUNK, HALF_D), jnp.float32),
        pltpu.VMEM((CHUNK, HALF_D), jnp.float32),
    ]
    + [pltpu.SemaphoreType.DMA] * 6,
)
def _sc_scatter(e3_hbm, y_hbm, out_hbm, z_sh, i0, i1, r0, r1, c0, c1, g0, g1,
                is0, is1, gs0, gs1, ss0, ss1):
    cid = lax.axis_index("c")
    sid = lax.axis_index("s")
    ybase = cid * N_NODES_K
    trash16 = ZTRASH + lax.iota(jnp.int32, 16)
    slots = [(i0, r0, c0, g0, is0, gs0, ss0), (i1, r1, c1, g1, is1, gs1, ss1)]

    def _fix(b):
        islot, rslot, cslot = slots[b][0], slots[b][1], slots[b][2]

        @pl.loop(0, CHUNK, step=16)
        def _(j):
            c16 = islot[1, pl.ds(j, 16)]
            r16 = islot[0, pl.ds(j, 16)]
            rslot[pl.ds(j, 16)] = r16 + ybase
            bad = (c16 >= N_NODES_K) | (c16 == r16)
            cslot[pl.ds(j, 16)] = jnp.where(bad, trash16, c16)

    def _idx(k, b):
        return pltpu.make_async_copy(
            e3_hbm.at[sid * NKS + k], slots[b][0], slots[b][4])

    def _gat(b):
        return pltpu.make_async_copy(
            y_hbm.at[slots[b][1]], slots[b][3], slots[b][5])

    def _scat(b):
        return pltpu.make_async_copy(
            slots[b][3], z_sh.at[slots[b][2]], slots[b][6])

    @pl.loop(0, CHUNK)
    def _(i):
        @pl.loop(0, HALF_D, step=16)
        def _(j):
            g0[i, pl.ds(j, 16)] = jnp.zeros((16,), jnp.float32)

    zb = sid * 632
    for part in range(4):
        pltpu.sync_copy(g0, z_sh.at[pl.ds(zb + part * CHUNK, CHUNK)])
    pltpu.sync_copy(
        g0.at[pl.ds(0, 120)], z_sh.at[pl.ds(zb + 512, 120)])
    plsc.subcore_barrier()

    for b in range(NB):
        _idx(b, b).start()

    @pl.loop(0, NKS, step=NB)
    def _(k2):
        for b in range(NB):
            k = k2 + b

            @pl.when(k2 > 0)
            def _():
                _scat(b).wait()

            _idx(k, b).wait()
            _fix(b)

            @pl.when(k + NB < NKS)
            def _():
                _idx(k + NB, b).start()

            _gat(b).start()

            if b > 0:
                _gat(b - 1).wait()
                _scat(b - 1).start(add=True)
            else:
                @pl.when(k2 > 0)
                def _():
                    _gat(NB - 1).wait()
                    _scat(NB - 1).start(add=True)

    _gat(NB - 1).wait()
    _scat(NB - 1).start(add=True)
    for b in range(NB):
        _scat(b).wait()

    plsc.subcore_barrier()
    pltpu.sync_copy(
        z_sh.at[pl.ds(zb, 632)], out_hbm.at[cid, pl.ds(zb, 632)]
    )


def _t1_body(d0, d1, x, y):
    deg = d0[...][:, 0:1] + d1[...][:, 0:1] + 1.0
    dinv = lax.rsqrt(deg)
    y[...] = dinv * x[...]


def _t2_write(d0, d1, z, yy, o):
    deg = d0[...][:, 0:1] + d1[...][:, 0:1] + 1.0
    dinv = lax.rsqrt(deg)
    lo = dinv * (z[...][0] + yy[...][0])
    hi = dinv * (z[...][1] + yy[...][1])
    o[...] = jnp.concatenate([lo, hi], axis=1)


def kernel(edge_index, input_feature):
    ei = edge_index.astype(jnp.int32)
    row, col = ei[0], ei[1]
    rowp = jnp.concatenate(
        [row, jnp.zeros((PAD_E,), jnp.int32)]).reshape(ROWS2D, CHUNK)
    colp = jnp.concatenate(
        [col, jnp.full((PAD_E,), N_NODES_K, jnp.int32)]).reshape(ROWS2D, CHUNK)
    e3 = jnp.stack([rowp, colp], axis=1)

    degp = _sc_degree(rowp, colp)
    d0 = degp[0, :N_NODES_K, :16]
    d1 = degp[1, :N_NODES_K, :16]

    nblk = N_NODES_K // BLK
    y = pl.pallas_call(
        _t1_body,
        out_shape=jax.ShapeDtypeStruct((2 * N_NODES_K, HALF_D), jnp.float32),
        grid=(2, nblk),
        in_specs=[
            pl.BlockSpec((BLK, 16), lambda h, i: (i, 0)),
            pl.BlockSpec((BLK, 16), lambda h, i: (i, 0)),
            pl.BlockSpec((BLK, HALF_D), lambda h, i: (i, h)),
        ],
        out_specs=pl.BlockSpec(
            (BLK, HALF_D), lambda h, i: (h * (N_NODES_K // BLK) + i, 0)),
    )(d0, d1, input_feature)

    zz = _sc_scatter(e3, y)
    z2 = zz[:, :N_NODES_K]
    y2 = y.reshape(2, N_NODES_K, HALF_D)

    out = pl.pallas_call(
        _t2_write,
        out_shape=jax.ShapeDtypeStruct((N_NODES_K, D_FEAT_K), jnp.float32),
        grid=(nblk,),
        in_specs=[
            pl.BlockSpec((BLK, 16), lambda i: (i, 0)),
            pl.BlockSpec((BLK, 16), lambda i: (i, 0)),
            pl.BlockSpec((2, BLK, HALF_D), lambda i: (0, i, 0)),
            pl.BlockSpec((2, BLK, HALF_D), lambda i: (0, i, 0)),
        ],
        out_specs=pl.BlockSpec((BLK, D_FEAT_K), lambda i: (i, 0)),
    )(d0, d1, z2, y2)
    return out


# speedup vs baseline: 1.0743x; 1.0743x over previous
import functools

import jax
import jax.numpy as jnp
from jax import lax
from jax.experimental import pallas as pl
from jax.experimental.pallas import tpu as pltpu
from jax.experimental.pallas import tpu_sc as plsc

N_NODES_K = 10000
N_EDGES_K = 160000
D_FEAT_K = 256
HALF_D = 128

CHUNK = 128
ROWS2D = 1280
PAD_E = ROWS2D * CHUNK - N_EDGES_K
NKS = ROWS2D // 16
NKD = ROWS2D // 32
NB = 2
ZROWS = 10112
ZTRASH = N_NODES_K
BLK = 400

_MESH = plsc.VectorSubcoreMesh(core_axis_name="c", subcore_axis_name="s")


@functools.partial(
    pl.kernel,
    out_type=jax.ShapeDtypeStruct((2, ZROWS, HALF_D), jnp.float32),
    mesh=_MESH,
    scratch_types=[
        pltpu.VMEM_SHARED((ZROWS, HALF_D), jnp.float32),
        pltpu.VMEM((NKD, CHUNK), jnp.int32),
        pltpu.VMEM((NKD, CHUNK), jnp.int32),
        pltpu.VMEM((CHUNK,), jnp.int32),
        pltpu.VMEM((CHUNK,), jnp.int32),
        pltpu.VMEM((CHUNK, HALF_D), jnp.float32),
        pltpu.VMEM((CHUNK, HALF_D), jnp.float32),
        pltpu.SemaphoreType.DMA,
        pltpu.SemaphoreType.DMA,
    ],
)
def _sc_degree(row_hbm, col_hbm, out_hbm, deg_sh, rbuf, cbuf, cs0, cs1, ones,
               zbuf, sem0, sem1):
    cid = lax.axis_index("c")
    sid = lax.axis_index("s")

    @pl.loop(0, CHUNK)
    def _(i):
        @pl.loop(0, HALF_D, step=16)
        def _(j):
            ones[i, pl.ds(j, 16)] = jnp.full((16,), 1.0, jnp.float32)
            zbuf[i, pl.ds(j, 16)] = jnp.zeros((16,), jnp.float32)

    base = sid * 632
    for part in range(4):
        pltpu.sync_copy(
            zbuf, deg_sh.at[pl.ds(base + part * CHUNK, CHUNK)])
    pltpu.sync_copy(
        zbuf.at[pl.ds(0, 120)], deg_sh.at[pl.ds(base + 512, 120)])
    plsc.subcore_barrier()

    base_r = cid * (16 * NKD) + sid * NKD
    pltpu.sync_copy(row_hbm.at[pl.ds(base_r, NKD)], rbuf)
    pltpu.sync_copy(col_hbm.at[pl.ds(base_r, NKD)], cbuf)

    trash16 = ZTRASH + lax.iota(jnp.int32, 16)
    slots = [(cs0, sem0), (cs1, sem1)]

    def _scat(b):
        cs, sem = slots[b]
        return pltpu.make_async_copy(ones, deg_sh.at[cs], sem)

    @pl.loop(0, NKD, step=NB)
    def _(k2):
        for b in range(NB):
            k = k2 + b
            cs, _sem = slots[b]

            @pl.when(k2 > 0)
            def _():
                _scat(b).wait()

            @pl.loop(0, CHUNK, step=16)
            def _(j):
                c16 = cbuf[k, pl.ds(j, 16)]
                r16 = rbuf[k, pl.ds(j, 16)]
                cs[pl.ds(j, 16)] = jnp.where(c16 == r16, trash16, c16)

            _scat(b).start(add=True)

    for b in range(NB):
        _scat(b).wait()

    plsc.subcore_barrier()
    pltpu.sync_copy(
        deg_sh.at[pl.ds(base, 632)],
        out_hbm.at[cid, pl.ds(base, 632)],
    )


@functools.partial(
    pl.kernel,
    out_type=jax.ShapeDtypeStruct((2, ZROWS, HALF_D), jnp.float32),
    mesh=_MESH,
    scratch_types=[
        pltpu.VMEM_SHARED((ZROWS, HALF_D), jnp.float32),
        pltpu.VMEM((2, CHUNK), jnp.int32),
        pltpu.VMEM((2, CHUNK), jnp.int32),
        pltpu.VMEM((CHUNK,), jnp.int32),
        pltpu.VMEM((CHUNK,), jnp.int32),
        pltpu.VMEM((CHUNK,), jnp.int32),
        pltpu.VMEM((CHUNK,), jnp.int32),
        pltpu.VMEM((CHUNK, HALF_D), jnp.float32),
        pltpu.VMEM((CHUNK, HALF_D), jnp.float32),
    ]
    + [pltpu.SemaphoreType.DMA] * 6,
)
def _sc_scatter(e3_hbm, y_hbm, out_hbm, z_sh, i0, i1, r0, r1, c0, c1, g0, g1,
                is0, is1, gs0, gs1, ss0, ss1):
    cid = lax.axis_index("c")
    sid = lax.axis_index("s")
    ybase = cid * N_NODES_K
    trash16 = ZTRASH + lax.iota(jnp.int32, 16)
    slots = [(i0, r0, c0, g0, is0, gs0, ss0), (i1, r1, c1, g1, is1, gs1, ss1)]

    def _fix(b):
        islot, rslot, cslot = slots[b][0], slots[b][1], slots[b][2]

        @pl.loop(0, CHUNK, step=16)
        def _(j):
            c16 = islot[1, pl.ds(j, 16)]
            r16 = islot[0, pl.ds(j, 16)]
            rslot[pl.ds(j, 16)] = r16 + ybase
            bad = (c16 >= N_NODES_K) | (c16 == r16)
            cslot[pl.ds(j, 16)] = jnp.where(bad, trash16, c16)

    def _idx(k, b):
        return pltpu.make_async_copy(
            e3_hbm.at[sid * NKS + k], slots[b][0], slots[b][4])

    def _gat(b):
        return pltpu.make_async_copy(
            y_hbm.at[slots[b][1]], slots[b][3], slots[b][5])

    def _scat(b):
        return pltpu.make_async_copy(
            slots[b][3], z_sh.at[slots[b][2]], slots[b][6])

    @pl.loop(0, CHUNK)
    def _(i):
        @pl.loop(0, HALF_D, step=16)
        def _(j):
            g0[i, pl.ds(j, 16)] = jnp.zeros((16,), jnp.float32)

    zb = sid * 632
    for part in range(4):
        pltpu.sync_copy(g0, z_sh.at[pl.ds(zb + part * CHUNK, CHUNK)])
    pltpu.sync_copy(
        g0.at[pl.ds(0, 120)], z_sh.at[pl.ds(zb + 512, 120)])
    plsc.subcore_barrier()

    for b in range(NB):
        _idx(b, b).start()

    @pl.loop(0, NKS, step=NB)
    def _(k2):
        for b in range(NB):
            k = k2 + b

            @pl.when(k2 > 0)
            def _():
                _scat(b).wait()

            _idx(k, b).wait()
            _fix(b)

            @pl.when(k + NB < NKS)
            def _():
                _idx(k + NB, b).start()

            _gat(b).start()

            if b > 0:
                _gat(b - 1).wait()
                _scat(b - 1).start(add=True)
            else:
                @pl.when(k2 > 0)
                def _():
                    _gat(NB - 1).wait()
                    _scat(NB - 1).start(add=True)

    _gat(NB - 1).wait()
    _scat(NB - 1).start(add=True)
    for b in range(NB):
        _scat(b).wait()

    plsc.subcore_barrier()
    pltpu.sync_copy(
        z_sh.at[pl.ds(zb, 632)], out_hbm.at[cid, pl.ds(zb, 632)]
    )


def _t1_body(d0, d1, x, y):
    deg = d0[...][0] + d1[...][0] + 1.0
    dinv = lax.rsqrt(deg)
    y[...] = dinv * x[...]


def _t2_write(d0, d1, z0, z1, y0, y1, o):
    deg = d0[...][0] + d1[...][0] + 1.0
    dinv = lax.rsqrt(deg)
    lo = dinv * (z0[...][0] + y0[...][0])
    hi = dinv * (z1[...][0] + y1[...][0])
    o[...] = jnp.concatenate([lo, hi], axis=1)


def kernel(edge_index, input_feature):
    ei = edge_index.astype(jnp.int32)
    row, col = ei[0], ei[1]
    rowp = jnp.concatenate(
        [row, jnp.zeros((PAD_E,), jnp.int32)]).reshape(ROWS2D, CHUNK)
    colp = jnp.concatenate(
        [col, jnp.full((PAD_E,), N_NODES_K, jnp.int32)]).reshape(ROWS2D, CHUNK)
    e3 = jnp.stack([rowp, colp], axis=1)

    degp = _sc_degree(rowp, colp)

    nblk = N_NODES_K // BLK
    dspec = [
        pl.BlockSpec((1, BLK, HALF_D), lambda h, i: (0, i, 0)),
        pl.BlockSpec((1, BLK, HALF_D), lambda h, i: (1, i, 0)),
    ]
    y = pl.pallas_call(
        _t1_body,
        out_shape=jax.ShapeDtypeStruct((2 * N_NODES_K, HALF_D), jnp.float32),
        grid=(2, nblk),
        in_specs=dspec + [
            pl.BlockSpec((BLK, HALF_D), lambda h, i: (i, h)),
        ],
        out_specs=pl.BlockSpec(
            (BLK, HALF_D), lambda h, i: (h * (N_NODES_K // BLK) + i, 0)),
    )(degp, degp, input_feature)

    zz = _sc_scatter(e3, y)
    y2 = y.reshape(2, N_NODES_K, HALF_D)

    out = pl.pallas_call(
        _t2_write,
        out_shape=jax.ShapeDtypeStruct((N_NODES_K, D_FEAT_K), jnp.float32),
        grid=(nblk,),
        in_specs=[
            pl.BlockSpec((1, BLK, HALF_D), lambda i: (0, i, 0)),
            pl.BlockSpec((1, BLK, HALF_D), lambda i: (1, i, 0)),
            pl.BlockSpec((1, BLK, HALF_D), lambda i: (0, i, 0)),
            pl.BlockSpec((1, BLK, HALF_D), lambda i: (1, i, 0)),
            pl.BlockSpec((1, BLK, HALF_D), lambda i: (0, i, 0)),
            pl.BlockSpec((1, BLK, HALF_D), lambda i: (1, i, 0)),
        ],
        out_specs=pl.BlockSpec((BLK, D_FEAT_K), lambda i: (i, 0)),
    )(degp, degp, zz, zz, y2, y2)
    return out


# speedup vs baseline: 1.2490x; 1.1626x over previous
import functools

import jax
import jax.numpy as jnp
from jax import lax
from jax.experimental import pallas as pl
from jax.experimental.pallas import tpu as pltpu
from jax.experimental.pallas import tpu_sc as plsc

N_NODES_K = 10000
N_EDGES_K = 160000
D_FEAT_K = 256
HALF_D = 128

CHUNK = 128
ROWS2D = 1280
PAD_E = ROWS2D * CHUNK - N_EDGES_K
NKS = ROWS2D // 16
NKD = ROWS2D // 32
NB = 2
ZROWS = 10112
ZTRASH = N_NODES_K
BLK = 400

_MESH = plsc.VectorSubcoreMesh(core_axis_name="c", subcore_axis_name="s")


@functools.partial(
    pl.kernel,
    out_type=jax.ShapeDtypeStruct((2, ZROWS, HALF_D), jnp.float32),
    mesh=_MESH,
    scratch_types=[
        pltpu.VMEM_SHARED((ZROWS, HALF_D), jnp.float32),
        pltpu.VMEM((NKD, CHUNK), jnp.int32),
        pltpu.VMEM((NKD, CHUNK), jnp.int32),
        pltpu.VMEM((CHUNK,), jnp.int32),
        pltpu.VMEM((CHUNK,), jnp.int32),
        pltpu.VMEM((CHUNK, HALF_D), jnp.float32),
        pltpu.VMEM((CHUNK, HALF_D), jnp.float32),
        pltpu.SemaphoreType.DMA,
        pltpu.SemaphoreType.DMA,
    ],
)
def _sc_degree(row_hbm, col_hbm, out_hbm, deg_sh, rbuf, cbuf, cs0, cs1, ones,
               zbuf, sem0, sem1):
    cid = lax.axis_index("c")
    sid = lax.axis_index("s")

    @pl.loop(0, CHUNK)
    def _(i):
        @pl.loop(0, HALF_D, step=16)
        def _(j):
            ones[i, pl.ds(j, 16)] = jnp.full((16,), 1.0, jnp.float32)
            zbuf[i, pl.ds(j, 16)] = jnp.zeros((16,), jnp.float32)

    base = sid * 632
    for part in range(4):
        pltpu.sync_copy(
            zbuf, deg_sh.at[pl.ds(base + part * CHUNK, CHUNK)])
    pltpu.sync_copy(
        zbuf.at[pl.ds(0, 120)], deg_sh.at[pl.ds(base + 512, 120)])
    plsc.subcore_barrier()

    base_r = cid * (16 * NKD) + sid * NKD
    pltpu.sync_copy(row_hbm.at[pl.ds(base_r, NKD)], rbuf)
    pltpu.sync_copy(col_hbm.at[pl.ds(base_r, NKD)], cbuf)

    trash16 = ZTRASH + lax.iota(jnp.int32, 16)
    slots = [(cs0, sem0), (cs1, sem1)]

    def _scat(b):
        cs, sem = slots[b]
        return pltpu.make_async_copy(ones, deg_sh.at[cs], sem)

    @pl.loop(0, NKD, step=NB)
    def _(k2):
        for b in range(NB):
            k = k2 + b
            cs, _sem = slots[b]

            @pl.when(k2 > 0)
            def _():
                _scat(b).wait()

            @pl.loop(0, CHUNK, step=16)
            def _(j):
                c16 = cbuf[k, pl.ds(j, 16)]
                r16 = rbuf[k, pl.ds(j, 16)]
                cs[pl.ds(j, 16)] = jnp.where(c16 == r16, trash16, c16)

            _scat(b).start(add=True)

    for b in range(NB):
        _scat(b).wait()

    plsc.subcore_barrier()
    pltpu.sync_copy(
        deg_sh.at[pl.ds(base, 632)],
        out_hbm.at[cid, pl.ds(base, 632)],
    )


@functools.partial(
    pl.kernel,
    out_type=jax.ShapeDtypeStruct((2, ZROWS, HALF_D), jnp.float32),
    mesh=_MESH,
    scratch_types=[
        pltpu.VMEM_SHARED((ZROWS, HALF_D), jnp.float32),
        pltpu.VMEM((2, CHUNK), jnp.int32),
        pltpu.VMEM((2, CHUNK), jnp.int32),
        pltpu.VMEM((CHUNK,), jnp.int32),
        pltpu.VMEM((CHUNK,), jnp.int32),
        pltpu.VMEM((CHUNK,), jnp.int32),
        pltpu.VMEM((CHUNK,), jnp.int32),
        pltpu.VMEM((CHUNK, HALF_D), jnp.float32),
        pltpu.VMEM((CHUNK, HALF_D), jnp.float32),
    ]
    + [pltpu.SemaphoreType.DMA] * 6,
)
def _sc_scatter(e3_hbm, y_hbm, out_hbm, z_sh, i0, i1, r0, r1, c0, c1, g0, g1,
                is0, is1, gs0, gs1, ss0, ss1):
    cid = lax.axis_index("c")
    sid = lax.axis_index("s")
    ybase = cid * N_NODES_K
    trash16 = ZTRASH + lax.iota(jnp.int32, 16)
    slots = [(i0, r0, c0, g0, is0, gs0, ss0), (i1, r1, c1, g1, is1, gs1, ss1)]

    def _fix(b):
        islot, rslot, cslot = slots[b][0], slots[b][1], slots[b][2]

        @pl.loop(0, CHUNK, step=16)
        def _(j):
            c16 = islot[1, pl.ds(j, 16)]
            r16 = islot[0, pl.ds(j, 16)]
            rslot[pl.ds(j, 16)] = r16 + ybase
            bad = (c16 >= N_NODES_K) | (c16 == r16)
            cslot[pl.ds(j, 16)] = jnp.where(bad, trash16, c16)

    def _idx(k, b):
        return pltpu.make_async_copy(
            e3_hbm.at[sid * NKS + k], slots[b][0], slots[b][4])

    def _gat(b):
        return pltpu.make_async_copy(
            y_hbm.at[slots[b][1]], slots[b][3], slots[b][5])

    def _scat(b):
        return pltpu.make_async_copy(
            slots[b][3], z_sh.at[slots[b][2]], slots[b][6])

    @pl.loop(0, CHUNK)
    def _(i):
        @pl.loop(0, HALF_D, step=16)
        def _(j):
            g0[i, pl.ds(j, 16)] = jnp.zeros((16,), jnp.float32)

    zb = sid * 632
    for part in range(4):
        pltpu.sync_copy(g0, z_sh.at[pl.ds(zb + part * CHUNK, CHUNK)])
    pltpu.sync_copy(
        g0.at[pl.ds(0, 120)], z_sh.at[pl.ds(zb + 512, 120)])
    plsc.subcore_barrier()

    for b in range(NB):
        _idx(b, b).start()

    @pl.loop(0, NKS, step=NB)
    def _(k2):
        for b in range(NB):
            k = k2 + b

            @pl.when(k2 > 0)
            def _():
                _scat(b).wait()

            _idx(k, b).wait()
            _fix(b)

            @pl.when(k + NB < NKS)
            def _():
                _idx(k + NB, b).start()

            _gat(b).start()

            if b > 0:
                _gat(b - 1).wait()
                _scat(b - 1).start(add=True)
            else:
                @pl.when(k2 > 0)
                def _():
                    _gat(NB - 1).wait()
                    _scat(NB - 1).start(add=True)

    _gat(NB - 1).wait()
    _scat(NB - 1).start(add=True)
    for b in range(NB):
        _scat(b).wait()

    plsc.subcore_barrier()
    pltpu.sync_copy(
        z_sh.at[pl.ds(zb, 632)], out_hbm.at[cid, pl.ds(zb, 632)]
    )


def _t1_body(d0, d1, x, y3, dv):
    deg = d0[...][0] + d1[...][0] + 1.0
    dinv = lax.rsqrt(deg)
    y3[0] = dinv * x[...][:, :HALF_D]
    y3[1] = dinv * x[...][:, HALF_D:]
    dv[...] = dinv


def _t2_write(dv, z, y, o):
    dinv = dv[...]
    lo = dinv * (z[...][0] + y[...][0])
    hi = dinv * (z[...][1] + y[...][1])
    o[...] = jnp.concatenate([lo, hi], axis=1)


def kernel(edge_index, input_feature):
    ei = edge_index.astype(jnp.int32)
    row, col = ei[0], ei[1]
    rowp = jnp.concatenate(
        [row, jnp.zeros((PAD_E,), jnp.int32)]).reshape(ROWS2D, CHUNK)
    colp = jnp.concatenate(
        [col, jnp.full((PAD_E,), N_NODES_K, jnp.int32)]).reshape(ROWS2D, CHUNK)
    e3 = jnp.stack([rowp, colp], axis=1)

    degp = _sc_degree(rowp, colp)

    nblk = N_NODES_K // BLK
    y3, dinvb = pl.pallas_call(
        _t1_body,
        out_shape=[
            jax.ShapeDtypeStruct((2, N_NODES_K, HALF_D), jnp.float32),
            jax.ShapeDtypeStruct((N_NODES_K, HALF_D), jnp.float32),
        ],
        grid=(nblk,),
        in_specs=[
            pl.BlockSpec((1, BLK, HALF_D), lambda i: (0, i, 0)),
            pl.BlockSpec((1, BLK, HALF_D), lambda i: (1, i, 0)),
            pl.BlockSpec((BLK, D_FEAT_K), lambda i: (i, 0)),
        ],
        out_specs=[
            pl.BlockSpec((2, BLK, HALF_D), lambda i: (0, i, 0)),
            pl.BlockSpec((BLK, HALF_D), lambda i: (i, 0)),
        ],
    )(degp, degp, input_feature)

    zz = _sc_scatter(e3, y3.reshape(2 * N_NODES_K, HALF_D))

    out = pl.pallas_call(
        _t2_write,
        out_shape=jax.ShapeDtypeStruct((N_NODES_K, D_FEAT_K), jnp.float32),
        grid=(nblk,),
        in_specs=[
            pl.BlockSpec((BLK, HALF_D), lambda i: (i, 0)),
            pl.BlockSpec((2, BLK, HALF_D), lambda i: (0, i, 0)),
            pl.BlockSpec((2, BLK, HALF_D), lambda i: (0, i, 0)),
        ],
        out_specs=pl.BlockSpec((BLK, D_FEAT_K), lambda i: (i, 0)),
    )(dinvb, zz, y3)
    return out


# speedup vs baseline: 1.2786x; 1.0237x over previous
import functools

import jax
import jax.numpy as jnp
from jax import lax
from jax.experimental import pallas as pl
from jax.experimental.pallas import tpu as pltpu
from jax.experimental.pallas import tpu_sc as plsc

N_NODES_K = 10000
N_EDGES_K = 160000
D_FEAT_K = 256
HALF_D = 128

CHUNK = 128
ROWS2D = 1280
PAD_E = ROWS2D * CHUNK - N_EDGES_K
NKS = ROWS2D // 16
NKD = ROWS2D // 32
NB = 2
ZROWS = 10112
ZTRASH = N_NODES_K
BLK = 400
DEGW = 64

_MESH = plsc.VectorSubcoreMesh(core_axis_name="c", subcore_axis_name="s")


@functools.partial(
    pl.kernel,
    out_type=jax.ShapeDtypeStruct((2, ZROWS, DEGW), jnp.float32),
    mesh=_MESH,
    scratch_types=[
        pltpu.VMEM_SHARED((ZROWS, DEGW), jnp.float32),
        pltpu.VMEM((NKD, CHUNK), jnp.int32),
        pltpu.VMEM((NKD, CHUNK), jnp.int32),
        pltpu.VMEM((CHUNK,), jnp.int32),
        pltpu.VMEM((CHUNK,), jnp.int32),
        pltpu.VMEM((CHUNK, DEGW), jnp.float32),
        pltpu.VMEM((CHUNK, DEGW), jnp.float32),
        pltpu.SemaphoreType.DMA,
        pltpu.SemaphoreType.DMA,
    ],
)
def _sc_degree(row_hbm, col_hbm, out_hbm, deg_sh, rbuf, cbuf, cs0, cs1, ones,
               zbuf, sem0, sem1):
    cid = lax.axis_index("c")
    sid = lax.axis_index("s")

    @pl.loop(0, CHUNK)
    def _(i):
        @pl.loop(0, DEGW, step=16)
        def _(j):
            ones[i, pl.ds(j, 16)] = jnp.full((16,), 1.0, jnp.float32)
            zbuf[i, pl.ds(j, 16)] = jnp.zeros((16,), jnp.float32)

    base = sid * 632
    for part in range(4):
        pltpu.sync_copy(
            zbuf, deg_sh.at[pl.ds(base + part * CHUNK, CHUNK)])
    pltpu.sync_copy(
        zbuf.at[pl.ds(0, 120)], deg_sh.at[pl.ds(base + 512, 120)])
    plsc.subcore_barrier()

    base_r = cid * (16 * NKD) + sid * NKD
    pltpu.sync_copy(row_hbm.at[pl.ds(base_r, NKD)], rbuf)
    pltpu.sync_copy(col_hbm.at[pl.ds(base_r, NKD)], cbuf)

    trash16 = ZTRASH + lax.iota(jnp.int32, 16)
    slots = [(cs0, sem0), (cs1, sem1)]

    def _scat(b):
        cs, sem = slots[b]
        return pltpu.make_async_copy(ones, deg_sh.at[cs], sem)

    @pl.loop(0, NKD, step=NB)
    def _(k2):
        for b in range(NB):
            k = k2 + b
            cs, _sem = slots[b]

            @pl.when(k2 > 0)
            def _():
                _scat(b).wait()

            @pl.loop(0, CHUNK, step=16)
            def _(j):
                c16 = cbuf[k, pl.ds(j, 16)]
                r16 = rbuf[k, pl.ds(j, 16)]
                cs[pl.ds(j, 16)] = jnp.where(c16 == r16, trash16, c16)

            _scat(b).start(add=True)

    for b in range(NB):
        _scat(b).wait()

    plsc.subcore_barrier()
    pltpu.sync_copy(
        deg_sh.at[pl.ds(base, 632)],
        out_hbm.at[cid, pl.ds(base, 632)],
    )


@functools.partial(
    pl.kernel,
    out_type=jax.ShapeDtypeStruct((2, ZROWS, HALF_D), jnp.float32),
    mesh=_MESH,
    scratch_types=[
        pltpu.VMEM_SHARED((ZROWS, HALF_D), jnp.float32),
        pltpu.VMEM((2, CHUNK), jnp.int32),
        pltpu.VMEM((2, CHUNK), jnp.int32),
        pltpu.VMEM((CHUNK,), jnp.int32),
        pltpu.VMEM((CHUNK,), jnp.int32),
        pltpu.VMEM((CHUNK,), jnp.int32),
        pltpu.VMEM((CHUNK,), jnp.int32),
        pltpu.VMEM((CHUNK, HALF_D), jnp.float32),
        pltpu.VMEM((CHUNK, HALF_D), jnp.float32),
    ]
    + [pltpu.SemaphoreType.DMA] * 6,
)
def _sc_scatter(e3_hbm, y_hbm, out_hbm, z_sh, i0, i1, r0, r1, c0, c1, g0, g1,
                is0, is1, gs0, gs1, ss0, ss1):
    cid = lax.axis_index("c")
    sid = lax.axis_index("s")
    ybase = cid * N_NODES_K
    trash16 = ZTRASH + lax.iota(jnp.int32, 16)
    slots = [(i0, r0, c0, g0, is0, gs0, ss0), (i1, r1, c1, g1, is1, gs1, ss1)]

    def _fix(b):
        islot, rslot, cslot = slots[b][0], slots[b][1], slots[b][2]

        @pl.loop(0, CHUNK, step=16)
        def _(j):
            c16 = islot[1, pl.ds(j, 16)]
            r16 = islot[0, pl.ds(j, 16)]
            rslot[pl.ds(j, 16)] = r16 + ybase
            bad = (c16 >= N_NODES_K) | (c16 == r16)
            cslot[pl.ds(j, 16)] = jnp.where(bad, trash16, c16)

    def _idx(k, b):
        return pltpu.make_async_copy(
            e3_hbm.at[sid * NKS + k], slots[b][0], slots[b][4])

    def _gat(b):
        return pltpu.make_async_copy(
            y_hbm.at[slots[b][1]], slots[b][3], slots[b][5])

    def _scat(b):
        return pltpu.make_async_copy(
            slots[b][3], z_sh.at[slots[b][2]], slots[b][6])

    @pl.loop(0, CHUNK)
    def _(i):
        @pl.loop(0, HALF_D, step=16)
        def _(j):
            g0[i, pl.ds(j, 16)] = jnp.zeros((16,), jnp.float32)

    zb = sid * 632
    for part in range(4):
        pltpu.sync_copy(g0, z_sh.at[pl.ds(zb + part * CHUNK, CHUNK)])
    pltpu.sync_copy(
        g0.at[pl.ds(0, 120)], z_sh.at[pl.ds(zb + 512, 120)])
    plsc.subcore_barrier()

    for b in range(NB):
        _idx(b, b).start()

    @pl.loop(0, NKS, step=NB)
    def _(k2):
        for b in range(NB):
            k = k2 + b

            @pl.when(k2 > 0)
            def _():
                _scat(b).wait()

            _idx(k, b).wait()
            _fix(b)

            @pl.when(k + NB < NKS)
            def _():
                _idx(k + NB, b).start()

            _gat(b).start()

            if b > 0:
                _gat(b - 1).wait()
                _scat(b - 1).start(add=True)
            else:
                @pl.when(k2 > 0)
                def _():
                    _gat(NB - 1).wait()
                    _scat(NB - 1).start(add=True)

    _gat(NB - 1).wait()
    _scat(NB - 1).start(add=True)
    for b in range(NB):
        _scat(b).wait()

    plsc.subcore_barrier()
    pltpu.sync_copy(
        z_sh.at[pl.ds(zb, 632)], out_hbm.at[cid, pl.ds(zb, 632)]
    )


def _t1_body(d0, d1, x, y3, dv):
    deg = d0[...][0][:, 0:1] + d1[...][0][:, 0:1] + 1.0
    dinv = lax.rsqrt(deg) * jnp.ones((1, HALF_D), jnp.float32)
    y3[0] = dinv * x[...][:, :HALF_D]
    y3[1] = dinv * x[...][:, HALF_D:]
    dv[...] = dinv


def _t2_write(dv, z, y, o):
    dinv = dv[...]
    lo = dinv * (z[...][0] + y[...][0])
    hi = dinv * (z[...][1] + y[...][1])
    o[...] = jnp.concatenate([lo, hi], axis=1)


def kernel(edge_index, input_feature):
    ei = edge_index.astype(jnp.int32)
    row, col = ei[0], ei[1]
    rowp = jnp.concatenate(
        [row, jnp.zeros((PAD_E,), jnp.int32)]).reshape(ROWS2D, CHUNK)
    colp = jnp.concatenate(
        [col, jnp.full((PAD_E,), N_NODES_K, jnp.int32)]).reshape(ROWS2D, CHUNK)
    e3 = jnp.stack([rowp, colp], axis=1)

    degp = _sc_degree(rowp, colp)

    nblk = N_NODES_K // BLK
    y3, dinvb = pl.pallas_call(
        _t1_body,
        out_shape=[
            jax.ShapeDtypeStruct((2, N_NODES_K, HALF_D), jnp.float32),
            jax.ShapeDtypeStruct((N_NODES_K, HALF_D), jnp.float32),
        ],
        grid=(nblk,),
        in_specs=[
            pl.BlockSpec((1, BLK, DEGW), lambda i: (0, i, 0)),
            pl.BlockSpec((1, BLK, DEGW), lambda i: (1, i, 0)),
            pl.BlockSpec((BLK, D_FEAT_K), lambda i: (i, 0)),
        ],
        out_specs=[
            pl.BlockSpec((2, BLK, HALF_D), lambda i: (0, i, 0)),
            pl.BlockSpec((BLK, HALF_D), lambda i: (i, 0)),
        ],
    )(degp, degp, input_feature)

    zz = _sc_scatter(e3, y3.reshape(2 * N_NODES_K, HALF_D))

    out = pl.pallas_call(
        _t2_write,
        out_shape=jax.ShapeDtypeStruct((N_NODES_K, D_FEAT_K), jnp.float32),
        grid=(nblk,),
        in_specs=[
            pl.BlockSpec((BLK, HALF_D), lambda i: (i, 0)),
            pl.BlockSpec((2, BLK, HALF_D), lambda i: (0, i, 0)),
            pl.BlockSpec((2, BLK, HALF_D), lambda i: (0, i, 0)),
        ],
        out_specs=pl.BlockSpec((BLK, D_FEAT_K), lambda i: (i, 0)),
    )(dinvb, zz, y3)
    return out
